# baseline (speedup 1.0000x reference)
import jax
import jax.numpy as jnp
from jax import lax
from jax.experimental import pallas as pl
from jax.experimental.pallas import tpu as pltpu
from jax.experimental.pallas import tpu_sc as plsc

N = 10000
E = 320000
D = 128
C = 40

NC, NS = 2, 16
NW = NC * NS
EPT = E // NW
CHUNK = 100
NCHUNK = EPT // CHUNK
HALF = NCHUNK // 2
RPW8 = 632
RPWL = N - (NS - 1) * RPW8

BN = 512
GRID = (N + BN - 1) // BN


def _sc_agg_body(h_hbm, zeros_hbm, edge_hbm, out_hbm,
                 edgv, rows0, rows1, agg_sh, sem0, sem1, ssem0, ssem1):
    c = lax.axis_index("c")
    s = lax.axis_index("s")
    w = s * NC + c

    def gather(j, buf, sem):
        return pltpu.async_copy(h_hbm.at[edgv.at[j, 0]], buf, sem)

    def scatter(j, buf, ssem):
        pltpu.async_copy(buf, agg_sh.at[edgv.at[j, 1]], ssem, add=True)

    def drain(buf, sem):
        pltpu.make_async_copy(h_hbm.at[edgv.at[0, 0]], buf, sem).wait()

    def drain_s(buf, ssem):
        pltpu.make_async_copy(buf, agg_sh.at[edgv.at[0, 1]], ssem).wait()

    pltpu.sync_copy(edge_hbm.at[w].at[0], edgv)
    gather(0, rows0, sem0)
    gather(1, rows1, sem1)

    def init_slice(src_hbm):
        @pl.when(s < NS - 1)
        def _():
            off = pl.multiple_of(s * RPW8, 8)
            pltpu.sync_copy(src_hbm.at[pl.ds(off, RPW8)],
                            agg_sh.at[pl.ds(off, RPW8)])

        @pl.when(s == NS - 1)
        def _():
            pltpu.sync_copy(src_hbm.at[pl.ds((NS - 1) * RPW8, RPWL)],
                            agg_sh.at[pl.ds((NS - 1) * RPW8, RPWL)])

    @pl.when(c == 0)
    def _():
        init_slice(h_hbm)

    @pl.when(c == 1)
    def _():
        init_slice(zeros_hbm)

    plsc.subcore_barrier()

    for p in range(2):
        if p == 1:
            pltpu.sync_copy(edge_hbm.at[w].at[1], edgv)
            gather(0, rows0, sem0)
            gather(1, rows1, sem1)

        def pair(i, carry):
            j2 = 2 * i
            drain(rows0, sem0)
            scatter(j2, rows0, ssem0)
            drain(rows1, sem1)
            scatter(j2 + 1, rows1, ssem1)
            drain_s(rows0, ssem0)
            gather(j2 + 2, rows0, sem0)
            drain_s(rows1, ssem1)
            gather(j2 + 3, rows1, sem1)
            return carry

        lax.fori_loop(0, HALF // 2 - 1, pair, 0, unroll=False)
        drain(rows0, sem0)
        scatter(HALF - 2, rows0, ssem0)
        drain(rows1, sem1)
        scatter(HALF - 1, rows1, ssem1)
        drain_s(rows0, ssem0)
        drain_s(rows1, ssem1)
    plsc.subcore_barrier()

    @pl.when(s < NS - 1)
    def _():
        off = pl.multiple_of(s * RPW8, 8)
        pltpu.sync_copy(agg_sh.at[pl.ds(off, RPW8)],
                        out_hbm.at[c].at[pl.ds(off, RPW8)])

    @pl.when(s == NS - 1)
    def _():
        pltpu.sync_copy(agg_sh.at[pl.ds((NS - 1) * RPW8, RPWL)],
                        out_hbm.at[c].at[pl.ds((NS - 1) * RPW8, RPWL)])


_sc_agg_cache = []


def _sc_agg(*args):
    if not _sc_agg_cache:
        _sc_agg_cache.append(pl.kernel(
            _sc_agg_body,
            out_type=jax.ShapeDtypeStruct((NC, N, D), jnp.float32),
            mesh=plsc.VectorSubcoreMesh(core_axis_name="c",
                                        subcore_axis_name="s",
                                        num_cores=NC, num_subcores=NS),
            scratch_types=[
                pltpu.VMEM((HALF, 2, CHUNK), jnp.int32),
                pltpu.VMEM((CHUNK, D), jnp.float32),
                pltpu.VMEM((CHUNK, D), jnp.float32),
                pltpu.VMEM_SHARED((N, D), jnp.float32),
                pltpu.SemaphoreType.DMA,
                pltpu.SemaphoreType.DMA,
                pltpu.SemaphoreType.DMA,
                pltpu.SemaphoreType.DMA,
            ],
        ))
    return _sc_agg_cache[0](*args)


def _mlpbn_body(p_ref, w1_ref, b1_ref, w2_ref, b2_ref, g_ref, bb_ref,
                out_ref, h2_s, stat_s):
    ph = pl.program_id(0)
    i = pl.program_id(1)

    @pl.when(ph == 0)
    def _():
        row = lax.broadcasted_iota(jnp.int32, (BN, 1), 0) + i * BN
        valid = row < N
        z = jnp.where(valid, p_ref[0] + p_ref[1], 0.0)
        z1 = jnp.maximum(
            jnp.dot(z, w1_ref[...], preferred_element_type=jnp.float32)
            + b1_ref[...], 0.0)
        h2 = jnp.maximum(
            jnp.dot(z1, w2_ref[...], preferred_element_type=jnp.float32)
            + b2_ref[...], 0.0)
        h2_s[pl.ds(i * BN, BN), :] = h2
        h2m = jnp.where(valid, h2, 0.0)
        new = jnp.concatenate(
            [jnp.sum(h2m, axis=0, keepdims=True),
             jnp.sum(h2m * h2m, axis=0, keepdims=True)], axis=0)

        @pl.when(i == 0)
        def _():
            stat_s[...] = new

        @pl.when(i > 0)
        def _():
            stat_s[...] = stat_s[...] + new

    @pl.when(ph == 1)
    def _():
        mu = stat_s[0:1, :] * (1.0 / N)
        var = stat_s[1:2, :] * (1.0 / N) - mu * mu
        inv = lax.rsqrt(var + 1e-5)
        out_ref[...] = (g_ref[...] * (h2_s[pl.ds(i * BN, BN), :] - mu)
                        * inv + bb_ref[...])


_mlpbn = pl.pallas_call(
    _mlpbn_body,
    grid=(2, GRID),
    in_specs=[
        pl.BlockSpec((NC, BN, D), lambda p, i: (0, i * (1 - p), 0)),
        pl.BlockSpec((D, D), lambda p, i: (0, 0)),
        pl.BlockSpec((1, D), lambda p, i: (0, 0)),
        pl.BlockSpec((D, D), lambda p, i: (0, 0)),
        pl.BlockSpec((1, D), lambda p, i: (0, 0)),
        pl.BlockSpec((1, D), lambda p, i: (0, 0)),
        pl.BlockSpec((1, D), lambda p, i: (0, 0)),
    ],
    out_specs=pl.BlockSpec((BN, D), lambda p, i: (i * p, 0)),
    out_shape=jax.ShapeDtypeStruct((N, D), jnp.float32),
    scratch_shapes=[
        pltpu.VMEM((GRID * BN, D), jnp.float32),
        pltpu.VMEM((2, D), jnp.float32),
    ],
    compiler_params=pltpu.CompilerParams(
        dimension_semantics=("arbitrary", "arbitrary")),
)


def _head_body(h_ref, w1_ref, b1_ref, w2_ref, b2_ref, out_ref):
    t = jnp.maximum(
        jnp.dot(h_ref[...], w1_ref[...], preferred_element_type=jnp.float32)
        + b1_ref[...], 0.0)
    s = (jnp.dot(t, w2_ref[...], preferred_element_type=jnp.float32)
         + b2_ref[...])
    m = jnp.max(s, axis=1, keepdims=True)
    lse = jnp.log(jnp.sum(jnp.exp(s - m), axis=1, keepdims=True)) + m
    out_ref[...] = s - lse


_head = pl.pallas_call(
    _head_body,
    grid=(GRID,),
    in_specs=[
        pl.BlockSpec((BN, D), lambda i: (i, 0)),
        pl.BlockSpec((D, D), lambda i: (0, 0)),
        pl.BlockSpec((1, D), lambda i: (0, 0)),
        pl.BlockSpec((D, D), lambda i: (0, 0)),
        pl.BlockSpec((1, D), lambda i: (0, 0)),
    ],
    out_specs=pl.BlockSpec((BN, D), lambda i: (i, 0)),
    out_shape=jax.ShapeDtypeStruct((N, D), jnp.float32),
)


def kernel(x, edge_index, params):
    edges = jnp.stack(
        [edge_index[0].astype(jnp.int32).reshape(NW, NCHUNK, CHUNK),
         edge_index[1].astype(jnp.int32).reshape(NW, NCHUNK, CHUNK)],
        axis=2).reshape(NW, 2, HALF, 2, CHUNK)
    zeros = jnp.zeros((N, D), jnp.float32)

    h = x
    for i in range(5):
        parts = _sc_agg(h, zeros, edges)
        h = _mlpbn(parts,
                   params[f'conv{i}_W1'],
                   params[f'conv{i}_b1'].reshape(1, D),
                   params[f'conv{i}_W2'],
                   params[f'conv{i}_b2'].reshape(1, D),
                   params[f'bn{i}_g'].reshape(1, D),
                   params[f'bn{i}_b'].reshape(1, D))

    w2p = jnp.zeros((D, D), jnp.float32).at[:, :C].set(params['fc2_W'])
    b2p = jnp.full((1, D), -1e30, jnp.float32).at[0, :C].set(params['fc2_b'])
    out = _head(h, params['fc1_W'], params['fc1_b'].reshape(1, D), w2p, b2p)
    return out[:, :C]


# speedup vs baseline: 1.2699x; 1.2699x over previous
import jax
import jax.numpy as jnp
from jax import lax
from jax.experimental import pallas as pl
from jax.experimental.pallas import tpu as pltpu
from jax.experimental.pallas import tpu_sc as plsc

N = 10000
E = 320000
D = 128
C = 40

NC, NS = 2, 16
NW = NC * NS
EPT = E // NW
CHUNK = 100
NCHUNK = EPT // CHUNK
HALF = NCHUNK // 2
RPW8 = 632
RPWL = N - (NS - 1) * RPW8

BN = 512
GRID = (N + BN - 1) // BN


def _sc_agg_body(h_hbm, zeros_hbm, src_hbm, dst_hbm, out_hbm,
                 srcv, dstv, rows0, rows1, agg_sh, sem0, sem1):
    c = lax.axis_index("c")
    s = lax.axis_index("s")
    w = s * NC + c

    def gather(j, buf, sem):
        return pltpu.async_copy(h_hbm.at[srcv.at[j]], buf, sem)

    def scatter(j, buf):
        pltpu.sync_copy(buf, agg_sh.at[dstv.at[j]], add=True)

    def drain(buf, sem):
        pltpu.make_async_copy(h_hbm.at[srcv.at[0]], buf, sem).wait()

    def stage(p):
        pltpu.sync_copy(src_hbm.at[w].at[p], srcv)
        pltpu.sync_copy(dst_hbm.at[w].at[p], dstv)

    stage(0)
    gather(0, rows0, sem0)
    gather(1, rows1, sem1)

    def init_slice(src_hbm):
        @pl.when(s < NS - 1)
        def _():
            off = pl.multiple_of(s * RPW8, 8)
            pltpu.sync_copy(src_hbm.at[pl.ds(off, RPW8)],
                            agg_sh.at[pl.ds(off, RPW8)])

        @pl.when(s == NS - 1)
        def _():
            pltpu.sync_copy(src_hbm.at[pl.ds((NS - 1) * RPW8, RPWL)],
                            agg_sh.at[pl.ds((NS - 1) * RPW8, RPWL)])

    @pl.when(c == 0)
    def _():
        init_slice(h_hbm)

    @pl.when(c == 1)
    def _():
        init_slice(zeros_hbm)

    plsc.subcore_barrier()

    for p in range(2):
        if p == 1:
            stage(1)
            gather(0, rows0, sem0)
            gather(1, rows1, sem1)

        def pair(i, carry):
            j2 = 2 * i
            drain(rows0, sem0)
            scatter(j2, rows0)
            gather(j2 + 2, rows0, sem0)
            drain(rows1, sem1)
            scatter(j2 + 1, rows1)
            gather(j2 + 3, rows1, sem1)
            return carry

        lax.fori_loop(0, HALF // 2 - 1, pair, 0, unroll=False)
        drain(rows0, sem0)
        scatter(HALF - 2, rows0)
        drain(rows1, sem1)
        scatter(HALF - 1, rows1)
    plsc.subcore_barrier()

    @pl.when(s < NS - 1)
    def _():
        off = pl.multiple_of(s * RPW8, 8)
        pltpu.sync_copy(agg_sh.at[pl.ds(off, RPW8)],
                        out_hbm.at[c].at[pl.ds(off, RPW8)])

    @pl.when(s == NS - 1)
    def _():
        pltpu.sync_copy(agg_sh.at[pl.ds((NS - 1) * RPW8, RPWL)],
                        out_hbm.at[c].at[pl.ds((NS - 1) * RPW8, RPWL)])


_sc_agg_cache = []


def _sc_agg(*args):
    if not _sc_agg_cache:
        _sc_agg_cache.append(pl.kernel(
            _sc_agg_body,
            out_type=jax.ShapeDtypeStruct((NC, N, D), jnp.float32),
            mesh=plsc.VectorSubcoreMesh(core_axis_name="c",
                                        subcore_axis_name="s",
                                        num_cores=NC, num_subcores=NS),
            scratch_types=[
                pltpu.VMEM((HALF, CHUNK), jnp.int32),
                pltpu.VMEM((HALF, CHUNK), jnp.int32),
                pltpu.VMEM((CHUNK, D), jnp.float32),
                pltpu.VMEM((CHUNK, D), jnp.float32),
                pltpu.VMEM_SHARED((N, D), jnp.float32),
                pltpu.SemaphoreType.DMA,
                pltpu.SemaphoreType.DMA,
            ],
        ))
    return _sc_agg_cache[0](*args)


def _layer_phase0(p_ref, w1_ref, b1_ref, w2_ref, b2_ref, i, h2_s, stat_s):
    row = lax.broadcasted_iota(jnp.int32, (BN, 1), 0) + i * BN
    valid = row < N
    z = jnp.where(valid, p_ref[0] + p_ref[1], 0.0)
    z1 = jnp.maximum(
        jnp.dot(z, w1_ref[...], preferred_element_type=jnp.float32)
        + b1_ref[...], 0.0)
    h2 = jnp.maximum(
        jnp.dot(z1, w2_ref[...], preferred_element_type=jnp.float32)
        + b2_ref[...], 0.0)
    h2_s[pl.ds(i * BN, BN), :] = h2
    h2m = jnp.where(valid, h2, 0.0)
    new = jnp.concatenate(
        [jnp.sum(h2m, axis=0, keepdims=True),
         jnp.sum(h2m * h2m, axis=0, keepdims=True)], axis=0)

    @pl.when(i == 0)
    def _():
        stat_s[...] = new

    @pl.when(i > 0)
    def _():
        stat_s[...] = stat_s[...] + new


def _bn_of(h2_blk, stat_s, g_ref, bb_ref):
    mu = stat_s[0:1, :] * (1.0 / N)
    var = stat_s[1:2, :] * (1.0 / N) - mu * mu
    inv = lax.rsqrt(var + 1e-5)
    return g_ref[...] * (h2_blk - mu) * inv + bb_ref[...]


def _mlpbn_body(p_ref, w1_ref, b1_ref, w2_ref, b2_ref, g_ref, bb_ref,
                out_ref, h2_s, stat_s):
    ph = pl.program_id(0)
    i = pl.program_id(1)

    @pl.when(ph == 0)
    def _():
        _layer_phase0(p_ref, w1_ref, b1_ref, w2_ref, b2_ref, i, h2_s, stat_s)

    @pl.when(ph == 1)
    def _():
        out_ref[...] = _bn_of(h2_s[pl.ds(i * BN, BN), :], stat_s,
                              g_ref, bb_ref)


def _mlpbn_head_body(p_ref, w1_ref, b1_ref, w2_ref, b2_ref, g_ref, bb_ref,
                     f1w_ref, f1b_ref, f2w_ref, f2b_ref,
                     out_ref, h2_s, stat_s):
    ph = pl.program_id(0)
    i = pl.program_id(1)

    @pl.when(ph == 0)
    def _():
        _layer_phase0(p_ref, w1_ref, b1_ref, w2_ref, b2_ref, i, h2_s, stat_s)

    @pl.when(ph == 1)
    def _():
        bnh = _bn_of(h2_s[pl.ds(i * BN, BN), :], stat_s, g_ref, bb_ref)
        t = jnp.maximum(
            jnp.dot(bnh, f1w_ref[...], preferred_element_type=jnp.float32)
            + f1b_ref[...], 0.0)
        sc = (jnp.dot(t, f2w_ref[...], preferred_element_type=jnp.float32)
              + f2b_ref[...])
        m = jnp.max(sc, axis=1, keepdims=True)
        lse = jnp.log(jnp.sum(jnp.exp(sc - m), axis=1, keepdims=True)) + m
        out_ref[...] = sc - lse


_W_SPECS = [
    pl.BlockSpec((D, D), lambda p, i: (0, 0)),
    pl.BlockSpec((1, D), lambda p, i: (0, 0)),
    pl.BlockSpec((D, D), lambda p, i: (0, 0)),
    pl.BlockSpec((1, D), lambda p, i: (0, 0)),
    pl.BlockSpec((1, D), lambda p, i: (0, 0)),
    pl.BlockSpec((1, D), lambda p, i: (0, 0)),
]

_LAYER_COMMON = dict(
    grid=(2, GRID),
    out_specs=pl.BlockSpec((BN, D), lambda p, i: (i * p, 0)),
    out_shape=jax.ShapeDtypeStruct((N, D), jnp.float32),
    scratch_shapes=[
        pltpu.VMEM((GRID * BN, D), jnp.float32),
        pltpu.VMEM((2, D), jnp.float32),
    ],
    compiler_params=pltpu.CompilerParams(
        dimension_semantics=("arbitrary", "arbitrary")),
)

_mlpbn = pl.pallas_call(
    _mlpbn_body,
    in_specs=[pl.BlockSpec((NC, BN, D), lambda p, i: (0, i * (1 - p), 0))]
    + _W_SPECS,
    **_LAYER_COMMON,
)

_mlpbn_head = pl.pallas_call(
    _mlpbn_head_body,
    in_specs=[pl.BlockSpec((NC, BN, D), lambda p, i: (0, i * (1 - p), 0))]
    + _W_SPECS
    + [
        pl.BlockSpec((D, D), lambda p, i: (0, 0)),
        pl.BlockSpec((1, D), lambda p, i: (0, 0)),
        pl.BlockSpec((D, D), lambda p, i: (0, 0)),
        pl.BlockSpec((1, D), lambda p, i: (0, 0)),
    ],
    **_LAYER_COMMON,
)


def kernel(x, edge_index, params):
    src = edge_index[0].astype(jnp.int32).reshape(NW, 2, HALF, CHUNK)
    dst = edge_index[1].astype(jnp.int32).reshape(NW, 2, HALF, CHUNK)
    zeros = jnp.zeros((N, D), jnp.float32)

    w2p = jnp.zeros((D, D), jnp.float32).at[:, :C].set(params['fc2_W'])
    b2p = jnp.full((1, D), -1e30, jnp.float32).at[0, :C].set(params['fc2_b'])

    h = x
    for i in range(5):
        parts = _sc_agg(h, zeros, src, dst)
        layer_args = (parts,
                      params[f'conv{i}_W1'],
                      params[f'conv{i}_b1'].reshape(1, D),
                      params[f'conv{i}_W2'],
                      params[f'conv{i}_b2'].reshape(1, D),
                      params[f'bn{i}_g'].reshape(1, D),
                      params[f'bn{i}_b'].reshape(1, D))
        if i < 4:
            h = _mlpbn(*layer_args)
        else:
            h = _mlpbn_head(*layer_args,
                            params['fc1_W'],
                            params['fc1_b'].reshape(1, D),
                            w2p, b2p)
    return h[:, :C]


# speedup vs baseline: 1.4020x; 1.1040x over previous
import jax
import jax.numpy as jnp
from jax import lax
from jax.experimental import pallas as pl
from jax.experimental.pallas import tpu as pltpu
from jax.experimental.pallas import tpu_sc as plsc

N = 10000
E = 320000
D = 128
C = 40

NC, NS = 2, 16
NW = NC * NS
EPT = E // NW
CHUNK = 100
NCHUNK = EPT // CHUNK
HALF = NCHUNK // 2
RPW8 = 632
RPWL = N - (NS - 1) * RPW8

BN = 2048
GRID = (N + BN - 1) // BN


def _sc_agg_body(h_hbm, zeros_hbm, src_hbm, dst_hbm, out_hbm,
                 srcv, dstv, rows0, rows1, agg_sh, sem0, sem1):
    c = lax.axis_index("c")
    s = lax.axis_index("s")
    w = s * NC + c

    def gather(j, buf, sem):
        return pltpu.async_copy(h_hbm.at[srcv.at[j]], buf, sem)

    def scatter(j, buf):
        pltpu.sync_copy(buf, agg_sh.at[dstv.at[j]], add=True)

    def drain(buf, sem):
        pltpu.make_async_copy(h_hbm.at[srcv.at[0]], buf, sem).wait()

    def stage(p):
        pltpu.sync_copy(src_hbm.at[w].at[p], srcv)
        pltpu.sync_copy(dst_hbm.at[w].at[p], dstv)

    stage(0)
    gather(0, rows0, sem0)
    gather(1, rows1, sem1)

    def init_slice(src_hbm):
        @pl.when(s < NS - 1)
        def _():
            off = pl.multiple_of(s * RPW8, 8)
            pltpu.sync_copy(src_hbm.at[pl.ds(off, RPW8)],
                            agg_sh.at[pl.ds(off, RPW8)])

        @pl.when(s == NS - 1)
        def _():
            pltpu.sync_copy(src_hbm.at[pl.ds((NS - 1) * RPW8, RPWL)],
                            agg_sh.at[pl.ds((NS - 1) * RPW8, RPWL)])

    @pl.when(c == 0)
    def _():
        init_slice(h_hbm)

    @pl.when(c == 1)
    def _():
        init_slice(zeros_hbm)

    plsc.subcore_barrier()

    for p in range(2):
        if p == 1:
            stage(1)
            gather(0, rows0, sem0)
            gather(1, rows1, sem1)

        def pair(i, carry):
            j2 = 2 * i
            drain(rows0, sem0)
            scatter(j2, rows0)
            gather(j2 + 2, rows0, sem0)
            drain(rows1, sem1)
            scatter(j2 + 1, rows1)
            gather(j2 + 3, rows1, sem1)
            return carry

        lax.fori_loop(0, HALF // 2 - 1, pair, 0, unroll=False)
        drain(rows0, sem0)
        scatter(HALF - 2, rows0)
        drain(rows1, sem1)
        scatter(HALF - 1, rows1)
    plsc.subcore_barrier()

    @pl.when(s < NS - 1)
    def _():
        off = pl.multiple_of(s * RPW8, 8)
        pltpu.sync_copy(agg_sh.at[pl.ds(off, RPW8)],
                        out_hbm.at[c].at[pl.ds(off, RPW8)])

    @pl.when(s == NS - 1)
    def _():
        pltpu.sync_copy(agg_sh.at[pl.ds((NS - 1) * RPW8, RPWL)],
                        out_hbm.at[c].at[pl.ds((NS - 1) * RPW8, RPWL)])


_sc_agg_cache = []


def _sc_agg(*args):
    if not _sc_agg_cache:
        _sc_agg_cache.append(pl.kernel(
            _sc_agg_body,
            out_type=jax.ShapeDtypeStruct((NC, N, D), jnp.float32),
            mesh=plsc.VectorSubcoreMesh(core_axis_name="c",
                                        subcore_axis_name="s",
                                        num_cores=NC, num_subcores=NS),
            scratch_types=[
                pltpu.VMEM((HALF, CHUNK), jnp.int32),
                pltpu.VMEM((HALF, CHUNK), jnp.int32),
                pltpu.VMEM((CHUNK, D), jnp.float32),
                pltpu.VMEM((CHUNK, D), jnp.float32),
                pltpu.VMEM_SHARED((N, D), jnp.float32),
                pltpu.SemaphoreType.DMA,
                pltpu.SemaphoreType.DMA,
            ],
        ))
    return _sc_agg_cache[0](*args)


def _layer_phase0(p_ref, w1_ref, b1_ref, w2_ref, b2_ref, i, h2_s, stat_s):
    row = lax.broadcasted_iota(jnp.int32, (BN, 1), 0) + i * BN
    valid = row < N
    z = jnp.where(valid, p_ref[0] + p_ref[1], 0.0)
    z1 = jnp.maximum(
        jnp.dot(z, w1_ref[...], preferred_element_type=jnp.float32)
        + b1_ref[...], 0.0)
    h2 = jnp.maximum(
        jnp.dot(z1, w2_ref[...], preferred_element_type=jnp.float32)
        + b2_ref[...], 0.0)
    h2_s[pl.ds(i * BN, BN), :] = h2
    h2m = jnp.where(valid, h2, 0.0)
    new = jnp.concatenate(
        [jnp.sum(h2m, axis=0, keepdims=True),
         jnp.sum(h2m * h2m, axis=0, keepdims=True)], axis=0)

    @pl.when(i == 0)
    def _():
        stat_s[...] = new

    @pl.when(i > 0)
    def _():
        stat_s[...] = stat_s[...] + new


def _bn_of(h2_blk, stat_s, g_ref, bb_ref):
    mu = stat_s[0:1, :] * (1.0 / N)
    var = stat_s[1:2, :] * (1.0 / N) - mu * mu
    inv = lax.rsqrt(var + 1e-5)
    return g_ref[...] * (h2_blk - mu) * inv + bb_ref[...]


def _mlpbn_body(p_ref, w1_ref, b1_ref, w2_ref, b2_ref, g_ref, bb_ref,
                out_ref, h2_s, stat_s):
    ph = pl.program_id(0)
    i = pl.program_id(1)

    @pl.when(ph == 0)
    def _():
        _layer_phase0(p_ref, w1_ref, b1_ref, w2_ref, b2_ref, i, h2_s, stat_s)

    @pl.when(ph == 1)
    def _():
        out_ref[...] = _bn_of(h2_s[pl.ds(i * BN, BN), :], stat_s,
                              g_ref, bb_ref)


def _mlpbn_head_body(p_ref, w1_ref, b1_ref, w2_ref, b2_ref, g_ref, bb_ref,
                     f1w_ref, f1b_ref, f2w_ref, f2b_ref,
                     out_ref, h2_s, stat_s):
    ph = pl.program_id(0)
    i = pl.program_id(1)

    @pl.when(ph == 0)
    def _():
        _layer_phase0(p_ref, w1_ref, b1_ref, w2_ref, b2_ref, i, h2_s, stat_s)

    @pl.when(ph == 1)
    def _():
        bnh = _bn_of(h2_s[pl.ds(i * BN, BN), :], stat_s, g_ref, bb_ref)
        t = jnp.maximum(
            jnp.dot(bnh, f1w_ref[...], preferred_element_type=jnp.float32)
            + f1b_ref[...], 0.0)
        sc = (jnp.dot(t, f2w_ref[...], preferred_element_type=jnp.float32)
              + f2b_ref[...])
        m = jnp.max(sc, axis=1, keepdims=True)
        lse = jnp.log(jnp.sum(jnp.exp(sc - m), axis=1, keepdims=True)) + m
        out_ref[...] = sc - lse


_W_SPECS = [
    pl.BlockSpec((D, D), lambda p, i: (0, 0)),
    pl.BlockSpec((1, D), lambda p, i: (0, 0)),
    pl.BlockSpec((D, D), lambda p, i: (0, 0)),
    pl.BlockSpec((1, D), lambda p, i: (0, 0)),
    pl.BlockSpec((1, D), lambda p, i: (0, 0)),
    pl.BlockSpec((1, D), lambda p, i: (0, 0)),
]

_LAYER_COMMON = dict(
    grid=(2, GRID),
    out_specs=pl.BlockSpec((BN, D), lambda p, i: (i * p, 0)),
    out_shape=jax.ShapeDtypeStruct((N, D), jnp.float32),
    scratch_shapes=[
        pltpu.VMEM((GRID * BN, D), jnp.float32),
        pltpu.VMEM((2, D), jnp.float32),
    ],
    compiler_params=pltpu.CompilerParams(
        dimension_semantics=("arbitrary", "arbitrary")),
)

_mlpbn = pl.pallas_call(
    _mlpbn_body,
    in_specs=[pl.BlockSpec((NC, BN, D), lambda p, i: (0, i * (1 - p), 0))]
    + _W_SPECS,
    **_LAYER_COMMON,
)

_mlpbn_head = pl.pallas_call(
    _mlpbn_head_body,
    in_specs=[pl.BlockSpec((NC, BN, D), lambda p, i: (0, i * (1 - p), 0))]
    + _W_SPECS
    + [
        pl.BlockSpec((D, D), lambda p, i: (0, 0)),
        pl.BlockSpec((1, D), lambda p, i: (0, 0)),
        pl.BlockSpec((D, D), lambda p, i: (0, 0)),
        pl.BlockSpec((1, D), lambda p, i: (0, 0)),
    ],
    **_LAYER_COMMON,
)


def kernel(x, edge_index, params):
    src = edge_index[0].astype(jnp.int32).reshape(NW, 2, HALF, CHUNK)
    dst = edge_index[1].astype(jnp.int32).reshape(NW, 2, HALF, CHUNK)
    zeros = jnp.zeros((N, D), jnp.float32)

    w2p = jnp.zeros((D, D), jnp.float32).at[:, :C].set(params['fc2_W'])
    b2p = jnp.full((1, D), -1e30, jnp.float32).at[0, :C].set(params['fc2_b'])

    h = x
    for i in range(5):
        parts = _sc_agg(h, zeros, src, dst)
        layer_args = (parts,
                      params[f'conv{i}_W1'],
                      params[f'conv{i}_b1'].reshape(1, D),
                      params[f'conv{i}_W2'],
                      params[f'conv{i}_b2'].reshape(1, D),
                      params[f'bn{i}_g'].reshape(1, D),
                      params[f'bn{i}_b'].reshape(1, D))
        if i < 4:
            h = _mlpbn(*layer_args)
        else:
            h = _mlpbn_head(*layer_args,
                            params['fc1_W'],
                            params['fc1_b'].reshape(1, D),
                            w2p, b2p)
    return h[:, :C]


# speedup vs baseline: 1.4422x; 1.0286x over previous
import jax
import jax.numpy as jnp
from jax import lax
from jax.experimental import pallas as pl
from jax.experimental.pallas import tpu as pltpu
from jax.experimental.pallas import tpu_sc as plsc

N = 10000
E = 320000
D = 128
C = 40

NC, NS = 2, 16
NW = NC * NS
EPT = E // NW
CHUNK = 125
NCHUNK = EPT // CHUNK
HALF = NCHUNK // 2
RPW8 = 632
RPWL = N - (NS - 1) * RPW8

BN = 2048
GRID = (N + BN - 1) // BN


def _sc_agg_body(h_hbm, zeros_hbm, src_hbm, dst_hbm, out_hbm,
                 srcv, dstv, rows0, rows1, agg_sh, sem0, sem1):
    c = lax.axis_index("c")
    s = lax.axis_index("s")
    w = s * NC + c

    def gather(j, buf, sem):
        return pltpu.async_copy(h_hbm.at[srcv.at[j]], buf, sem)

    def scatter(j, buf):
        pltpu.sync_copy(buf, agg_sh.at[dstv.at[j]], add=True)

    def drain(buf, sem):
        pltpu.make_async_copy(h_hbm.at[srcv.at[0]], buf, sem).wait()

    def stage(p):
        pltpu.sync_copy(src_hbm.at[w].at[p], srcv)
        pltpu.sync_copy(dst_hbm.at[w].at[p], dstv)

    stage(0)
    gather(0, rows0, sem0)
    gather(1, rows1, sem1)

    def init_slice(src_hbm):
        @pl.when(s < NS - 1)
        def _():
            off = pl.multiple_of(s * RPW8, 8)
            pltpu.sync_copy(src_hbm.at[pl.ds(off, RPW8)],
                            agg_sh.at[pl.ds(off, RPW8)])

        @pl.when(s == NS - 1)
        def _():
            pltpu.sync_copy(src_hbm.at[pl.ds((NS - 1) * RPW8, RPWL)],
                            agg_sh.at[pl.ds((NS - 1) * RPW8, RPWL)])

    @pl.when(c == 0)
    def _():
        init_slice(h_hbm)

    @pl.when(c == 1)
    def _():
        init_slice(zeros_hbm)

    plsc.subcore_barrier()

    for p in range(2):
        if p == 1:
            stage(1)
            gather(0, rows0, sem0)
            gather(1, rows1, sem1)

        def pair(i, carry):
            j2 = 2 * i
            drain(rows0, sem0)
            scatter(j2, rows0)
            gather(j2 + 2, rows0, sem0)
            drain(rows1, sem1)
            scatter(j2 + 1, rows1)
            gather(j2 + 3, rows1, sem1)
            return carry

        lax.fori_loop(0, HALF // 2 - 1, pair, 0, unroll=False)
        drain(rows0, sem0)
        scatter(HALF - 2, rows0)
        drain(rows1, sem1)
        scatter(HALF - 1, rows1)
    plsc.subcore_barrier()

    @pl.when(s < NS - 1)
    def _():
        off = pl.multiple_of(s * RPW8, 8)
        pltpu.sync_copy(agg_sh.at[pl.ds(off, RPW8)],
                        out_hbm.at[c].at[pl.ds(off, RPW8)])

    @pl.when(s == NS - 1)
    def _():
        pltpu.sync_copy(agg_sh.at[pl.ds((NS - 1) * RPW8, RPWL)],
                        out_hbm.at[c].at[pl.ds((NS - 1) * RPW8, RPWL)])


_sc_agg_cache = []


def _sc_agg(*args):
    if not _sc_agg_cache:
        _sc_agg_cache.append(pl.kernel(
            _sc_agg_body,
            out_type=jax.ShapeDtypeStruct((NC, N, D), jnp.float32),
            mesh=plsc.VectorSubcoreMesh(core_axis_name="c",
                                        subcore_axis_name="s",
                                        num_cores=NC, num_subcores=NS),
            scratch_types=[
                pltpu.VMEM((HALF, CHUNK), jnp.int32),
                pltpu.VMEM((HALF, CHUNK), jnp.int32),
                pltpu.VMEM((CHUNK, D), jnp.float32),
                pltpu.VMEM((CHUNK, D), jnp.float32),
                pltpu.VMEM_SHARED((N, D), jnp.float32),
                pltpu.SemaphoreType.DMA,
                pltpu.SemaphoreType.DMA,
            ],
        ))
    return _sc_agg_cache[0](*args)


def _layer_phase0(p_ref, w1_ref, b1_ref, w2_ref, b2_ref, i, h2_s, stat_s):
    row = lax.broadcasted_iota(jnp.int32, (BN, 1), 0) + i * BN
    valid = row < N
    z = jnp.where(valid, p_ref[0] + p_ref[1], 0.0)
    z1 = jnp.maximum(
        jnp.dot(z, w1_ref[...], preferred_element_type=jnp.float32)
        + b1_ref[...], 0.0)
    h2 = jnp.maximum(
        jnp.dot(z1, w2_ref[...], preferred_element_type=jnp.float32)
        + b2_ref[...], 0.0)
    h2_s[pl.ds(i * BN, BN), :] = h2
    h2m = jnp.where(valid, h2, 0.0)
    new = jnp.concatenate(
        [jnp.sum(h2m, axis=0, keepdims=True),
         jnp.sum(h2m * h2m, axis=0, keepdims=True)], axis=0)

    @pl.when(i == 0)
    def _():
        stat_s[...] = new

    @pl.when(i > 0)
    def _():
        stat_s[...] = stat_s[...] + new


def _bn_of(h2_blk, stat_s, g_ref, bb_ref):
    mu = stat_s[0:1, :] * (1.0 / N)
    var = stat_s[1:2, :] * (1.0 / N) - mu * mu
    inv = lax.rsqrt(var + 1e-5)
    return g_ref[...] * (h2_blk - mu) * inv + bb_ref[...]


def _mlpbn_body(p_ref, w1_ref, b1_ref, w2_ref, b2_ref, g_ref, bb_ref,
                out_ref, h2_s, stat_s):
    ph = pl.program_id(0)
    i = pl.program_id(1)

    @pl.when(ph == 0)
    def _():
        _layer_phase0(p_ref, w1_ref, b1_ref, w2_ref, b2_ref, i, h2_s, stat_s)

    @pl.when(ph == 1)
    def _():
        out_ref[...] = _bn_of(h2_s[pl.ds(i * BN, BN), :], stat_s,
                              g_ref, bb_ref)


def _mlpbn_head_body(p_ref, w1_ref, b1_ref, w2_ref, b2_ref, g_ref, bb_ref,
                     f1w_ref, f1b_ref, f2w_ref, f2b_ref,
                     out_ref, h2_s, stat_s):
    ph = pl.program_id(0)
    i = pl.program_id(1)

    @pl.when(ph == 0)
    def _():
        _layer_phase0(p_ref, w1_ref, b1_ref, w2_ref, b2_ref, i, h2_s, stat_s)

    @pl.when(ph == 1)
    def _():
        bnh = _bn_of(h2_s[pl.ds(i * BN, BN), :], stat_s, g_ref, bb_ref)
        t = jnp.maximum(
            jnp.dot(bnh, f1w_ref[...], preferred_element_type=jnp.float32)
            + f1b_ref[...], 0.0)
        sc = (jnp.dot(t, f2w_ref[...], preferred_element_type=jnp.float32)
              + f2b_ref[...])
        m = jnp.max(sc, axis=1, keepdims=True)
        lse = jnp.log(jnp.sum(jnp.exp(sc - m), axis=1, keepdims=True)) + m
        out_ref[...] = sc - lse


_W_SPECS = [
    pl.BlockSpec((D, D), lambda p, i: (0, 0)),
    pl.BlockSpec((1, D), lambda p, i: (0, 0)),
    pl.BlockSpec((D, D), lambda p, i: (0, 0)),
    pl.BlockSpec((1, D), lambda p, i: (0, 0)),
    pl.BlockSpec((1, D), lambda p, i: (0, 0)),
    pl.BlockSpec((1, D), lambda p, i: (0, 0)),
]

_LAYER_COMMON = dict(
    grid=(2, GRID),
    out_specs=pl.BlockSpec((BN, D), lambda p, i: (i * p, 0)),
    out_shape=jax.ShapeDtypeStruct((N, D), jnp.float32),
    scratch_shapes=[
        pltpu.VMEM((GRID * BN, D), jnp.float32),
        pltpu.VMEM((2, D), jnp.float32),
    ],
    compiler_params=pltpu.CompilerParams(
        dimension_semantics=("arbitrary", "arbitrary")),
)

_mlpbn = pl.pallas_call(
    _mlpbn_body,
    in_specs=[pl.BlockSpec((NC, BN, D), lambda p, i: (0, i * (1 - p), 0))]
    + _W_SPECS,
    **_LAYER_COMMON,
)

_mlpbn_head = pl.pallas_call(
    _mlpbn_head_body,
    in_specs=[pl.BlockSpec((NC, BN, D), lambda p, i: (0, i * (1 - p), 0))]
    + _W_SPECS
    + [
        pl.BlockSpec((D, D), lambda p, i: (0, 0)),
        pl.BlockSpec((1, D), lambda p, i: (0, 0)),
        pl.BlockSpec((D, D), lambda p, i: (0, 0)),
        pl.BlockSpec((1, D), lambda p, i: (0, 0)),
    ],
    **_LAYER_COMMON,
)


def kernel(x, edge_index, params):
    src = edge_index[0].astype(jnp.int32).reshape(NW, 2, HALF, CHUNK)
    dst = edge_index[1].astype(jnp.int32).reshape(NW, 2, HALF, CHUNK)
    zeros = jnp.zeros((N, D), jnp.float32)

    w2p = jnp.zeros((D, D), jnp.float32).at[:, :C].set(params['fc2_W'])
    b2p = jnp.full((1, D), -1e30, jnp.float32).at[0, :C].set(params['fc2_b'])

    h = x
    for i in range(5):
        parts = _sc_agg(h, zeros, src, dst)
        layer_args = (parts,
                      params[f'conv{i}_W1'],
                      params[f'conv{i}_b1'].reshape(1, D),
                      params[f'conv{i}_W2'],
                      params[f'conv{i}_b2'].reshape(1, D),
                      params[f'bn{i}_g'].reshape(1, D),
                      params[f'bn{i}_b'].reshape(1, D))
        if i < 4:
            h = _mlpbn(*layer_args)
        else:
            h = _mlpbn_head(*layer_args,
                            params['fc1_W'],
                            params['fc1_b'].reshape(1, D),
                            w2p, b2p)
    return h[:, :C]
